# Initial kernel scaffold; baseline (speedup 1.0000x reference)
import jax
import jax.numpy as jnp
from jax.experimental import pallas as pl


def kernel(input_ids, token_type_ids, word_emb, pos_emb, type_emb, gamma, beta):
    raise NotImplementedError("write your pallas kernel here")


# speedup vs baseline: 3.7089x; 3.7089x over previous
import functools

import jax
import jax.numpy as jnp
from jax import lax
from jax.experimental import pallas as pl
from jax.experimental.pallas import tpu as pltpu
from jax.experimental.pallas import tpu_sc as plsc

VOCAB = 30522
MAX_POS = 512
HIDDEN = 64
B = 1024
L = 512
EPS = 1e-12

NC = 2
NS = 16
NW = NC * NS

TOKENS = B * L
CHUNK = 128
CHUNKS = TOKENS // CHUNK
CHUNKS_PER_W = CHUNKS // NW
PARTS = L // CHUNK


def _ln_rows(mesh):
    @functools.partial(
        pl.kernel,
        mesh=mesh,
        compiler_params=pltpu.CompilerParams(use_tc_tiling_on_sc=False),
        out_type=jax.ShapeDtypeStruct((TOKENS, HIDDEN), jnp.float32),
        scratch_types=[
            pltpu.VMEM((MAX_POS, HIDDEN), jnp.float32),
            pltpu.VMEM((2, HIDDEN), jnp.float32),
            pltpu.VMEM((2, HIDDEN), jnp.float32),
            pltpu.VMEM((CHUNK,), jnp.int32),
            pltpu.VMEM((CHUNK,), jnp.int32),
            pltpu.VMEM((CHUNK, HIDDEN), jnp.float32),
            pltpu.SemaphoreType.DMA,
        ],
    )
    def k(word_hbm, ids_hbm, tt_hbm, pos_hbm, type_hbm, gb_hbm, out_hbm,
          posv, typev, gbv, idxv, ttv, rowsv, sem):
        wid = lax.axis_index("s") * NC + lax.axis_index("c")

        pltpu.sync_copy(pos_hbm, posv)
        pltpu.sync_copy(type_hbm, typev)
        pltpu.sync_copy(gb_hbm, gbv)

        nv = HIDDEN // 16
        lane = lax.iota(jnp.int32, 16)
        rots = [(lane + k) & 15 for k in (8, 4, 2, 1)]

        dnums = lax.GatherDimensionNumbers(
            offset_dims=(), collapsed_slice_dims=(0,), start_index_map=(0,))

        def allr(v):
            for ridx in rots:
                v = v + lax.gather(
                    v, ridx[:, None], dnums, (1,),
                    mode=lax.GatherScatterMode.PROMISE_IN_BOUNDS)
            return v

        t0 = [typev[0, pl.ds(16 * h, 16)] for h in range(nv)]
        t1 = [typev[1, pl.ds(16 * h, 16)] for h in range(nv)]
        dt = [t1[h] - t0[h] for h in range(nv)]
        gv = [gbv[0, pl.ds(16 * h, 16)] for h in range(nv)]
        bv = [gbv[1, pl.ds(16 * h, 16)] for h in range(nv)]

        def chunk_body(kk, carry):
            r = wid * CHUNKS_PER_W + kk
            pltpu.sync_copy(ids_hbm.at[r], idxv)
            pltpu.sync_copy(tt_hbm.at[r], ttv)
            pltpu.async_copy(word_hbm.at[idxv], rowsv, sem).wait()
            pbase = (kk % PARTS) * CHUNK

            def grp_body(g, c2):
                tb = g * 16
                ttg = ttv[pl.ds(tb, 16)].astype(jnp.float32)
                for j in range(16):
                    t = tb + j
                    ttf = ttg[j]
                    pt = pbase + t
                    x = []
                    for h in range(nv):
                        w = rowsv[t, pl.ds(16 * h, 16)]
                        p = posv[pt, pl.ds(16 * h, 16)]
                        x.append(w + p + t0[h] + ttf * dt[h])
                    s = (x[0] + x[1]) + (x[2] + x[3])
                    mean = allr(s) * (1.0 / HIDDEN)
                    q = [xi * xi for xi in x]
                    qs = (q[0] + q[1]) + (q[2] + q[3])
                    var = (allr(qs)[0] * (1.0 / HIDDEN)
                           - mean[0] * mean[0] + EPS)
                    iv = lax.bitcast_convert_type(var, jnp.int32)
                    iv = jnp.int32(0x5F3759DF) - lax.shift_right_arithmetic(
                        iv, jnp.int32(1))
                    y = lax.bitcast_convert_type(iv, jnp.float32)
                    hvar = 0.5 * var
                    y = y * (1.5 - hvar * y * y)
                    y = y * (1.5 - hvar * y * y)
                    y = y * (1.5 - hvar * y * y)
                    for h in range(nv):
                        rowsv[t, pl.ds(16 * h, 16)] = (
                            (x[h] - mean) * y * gv[h] + bv[h])
                return c2

            lax.fori_loop(0, CHUNK // 16, grp_body, 0)
            pltpu.sync_copy(rowsv, out_hbm.at[pl.ds(r * CHUNK, CHUNK)])
            return carry

        lax.fori_loop(0, CHUNKS_PER_W, chunk_body, 0)

    return k


def kernel(input_ids, token_type_ids, word_emb, pos_emb, type_emb, gamma,
           beta):
    ids2d = input_ids.reshape(CHUNKS, CHUNK).astype(jnp.int32)
    tt2d = token_type_ids.reshape(CHUNKS, CHUNK).astype(jnp.int32)
    gb = jnp.stack([gamma, beta]).astype(jnp.float32)
    mesh = plsc.VectorSubcoreMesh(core_axis_name="c", subcore_axis_name="s")
    out = _ln_rows(mesh)(word_emb, ids2d, tt2d, pos_emb, type_emb, gb)
    return out.reshape(B, L, HIDDEN)


# speedup vs baseline: 4.3707x; 1.1784x over previous
import functools

import jax
import jax.numpy as jnp
from jax import lax
from jax.experimental import pallas as pl
from jax.experimental.pallas import tpu as pltpu
from jax.experimental.pallas import tpu_sc as plsc

VOCAB = 30522
MAX_POS = 512
HIDDEN = 64
B = 1024
L = 512
EPS = 1e-12

NC = 2
NS = 16
NW = NC * NS

TOKENS = B * L
CHUNK = 128
CHUNKS = TOKENS // CHUNK
PARTS = L // CHUNK
NSEQ_W = B // NW


def _ln_rows(mesh):
    @functools.partial(
        pl.kernel,
        mesh=mesh,
        compiler_params=pltpu.CompilerParams(use_tc_tiling_on_sc=False),
        out_type=jax.ShapeDtypeStruct((TOKENS, HIDDEN), jnp.float32),
        scratch_types=(
            [
                pltpu.VMEM((MAX_POS, HIDDEN), jnp.float32),
                pltpu.VMEM((2, HIDDEN), jnp.float32),
                pltpu.VMEM((2, HIDDEN), jnp.float32),
                pltpu.VMEM((PARTS, CHUNK), jnp.int32),
                pltpu.VMEM((PARTS, CHUNK), jnp.int32),
                pltpu.VMEM((PARTS, CHUNK), jnp.int32),
                pltpu.VMEM((PARTS, CHUNK), jnp.int32),
                pltpu.VMEM((2 * PARTS, CHUNK, HIDDEN), jnp.float32),
            ]
            + [pltpu.SemaphoreType.DMA] * (2 * PARTS)
            + [pltpu.SemaphoreType.DMA] * (2 * PARTS)
            + [pltpu.SemaphoreType.DMA] * 2
        ),
    )
    def k(word_hbm, ids_hbm, tt_hbm, pos_hbm, type_hbm, gb_hbm, out_hbm,
          posv, typev, gbv, idxa, idxb, tta, ttb, rowsv, *sems):
        idxv = (idxa, idxb)
        ttv = (tta, ttb)
        gsem = sems[:2 * PARTS]
        ssem = sems[2 * PARTS:4 * PARTS]
        isem, tsem = sems[4 * PARTS], sems[4 * PARTS + 1]
        wid = lax.axis_index("s") * NC + lax.axis_index("c")
        cbase = wid * NSEQ_W * PARTS

        pltpu.sync_copy(pos_hbm, posv)
        pltpu.sync_copy(type_hbm, typev)
        pltpu.sync_copy(gb_hbm, gbv)

        nv = HIDDEN // 16
        lane = lax.iota(jnp.int32, 16)
        rots = [(lane + kk) & 15 for kk in (8, 4, 2, 1)]
        dnums = lax.GatherDimensionNumbers(
            offset_dims=(), collapsed_slice_dims=(0,), start_index_map=(0,))

        def allr(v):
            for ridx in rots:
                v = v + lax.gather(
                    v, ridx[:, None], dnums, (1,),
                    mode=lax.GatherScatterMode.PROMISE_IN_BOUNDS)
            return v

        t0 = [typev[0, pl.ds(16 * h, 16)] for h in range(nv)]
        t1 = [typev[1, pl.ds(16 * h, 16)] for h in range(nv)]
        dt = [t1[h] - t0[h] for h in range(nv)]
        gv = [gbv[0, pl.ds(16 * h, 16)] for h in range(nv)]
        bv = [gbv[1, pl.ds(16 * h, 16)] for h in range(nv)]

        def fold_body(p, c):
            for h in range(nv):
                posv[p, pl.ds(16 * h, 16)] = (
                    posv[p, pl.ds(16 * h, 16)] + t0[h])
            return c
        lax.fori_loop(0, MAX_POS, fold_body, 0)

        def fire_gathers(q, par):
            for j in range(PARTS):
                pltpu.make_async_copy(
                    word_hbm.at[idxv[par].at[j]],
                    rowsv.at[par * PARTS + j], gsem[par * PARTS + j]).start()

        def drain_stores(par):
            for j in range(PARTS):
                pltpu.make_async_copy(
                    rowsv.at[par * PARTS + j],
                    out_hbm.at[pl.ds((par * PARTS + j) * CHUNK, CHUNK)],
                    ssem[par * PARTS + j]).wait()

        def compute_seq(q, par):
            rb = par * PARTS

            def chunk_body(j, c):
                pbase = j * CHUNK

                def grp_body(g, c2):
                    tgt = g * 16
                    ttg = ttv[par][j, pl.ds(tgt, 16)].astype(jnp.float32)
                    for jj in range(16):
                        t = tgt + jj
                        ttf = ttg[jj]
                        pt = pbase + t
                        x = []
                        for h in range(nv):
                            w = rowsv[rb + j, t, pl.ds(16 * h, 16)]
                            p = posv[pt, pl.ds(16 * h, 16)]
                            x.append(w + p + ttf * dt[h])
                        s = (x[0] + x[1]) + (x[2] + x[3])
                        mean = allr(s) * (1.0 / HIDDEN)
                        qq = [xi * xi for xi in x]
                        qs = (qq[0] + qq[1]) + (qq[2] + qq[3])
                        var = (allr(qs)[0] * (1.0 / HIDDEN)
                               - mean[0] * mean[0] + EPS)
                        iv = lax.bitcast_convert_type(var, jnp.int32)
                        iv = (jnp.int32(0x5F3759DF)
                              - lax.shift_right_arithmetic(iv, jnp.int32(1)))
                        y = lax.bitcast_convert_type(iv, jnp.float32)
                        hvar = 0.5 * var
                        y = y * (1.5 - hvar * y * y)
                        y = y * (1.5 - hvar * y * y)
                        y = y * (1.5 - hvar * y * y)
                        for h in range(nv):
                            rowsv[rb + j, t, pl.ds(16 * h, 16)] = (
                                (x[h] - mean) * y * gv[h] + bv[h])
                    return c2

                lax.fori_loop(0, CHUNK // 16, grp_body, 0)
                return c

            lax.fori_loop(0, PARTS, chunk_body, 0)

        def seq_step(q, par):
            other = 1 - par
            r0 = cbase + q * PARTS

            @pl.when(q + 1 < NSEQ_W)
            def _():
                pltpu.make_async_copy(
                    ids_hbm.at[pl.ds(r0, PARTS)], idxv[other],
                    isem).wait()
                pltpu.make_async_copy(
                    tt_hbm.at[pl.ds(r0, PARTS)], ttv[other],
                    tsem).wait()

                @pl.when(q > 0)
                def _():
                    drain_stores(other)
                fire_gathers(q + 1, other)

            for j in range(PARTS):
                pltpu.make_async_copy(
                    word_hbm.at[idxv[par].at[j]],
                    rowsv.at[par * PARTS + j], gsem[par * PARTS + j]).wait()

            compute_seq(q, par)

            @pl.when(q + 2 < NSEQ_W)
            def _():
                rn = r0 + 2 * PARTS
                pltpu.make_async_copy(
                    ids_hbm.at[pl.ds(rn, PARTS)], idxv[par], isem).start()
                pltpu.make_async_copy(
                    tt_hbm.at[pl.ds(rn, PARTS)], ttv[par], tsem).start()

            for j in range(PARTS):
                pltpu.make_async_copy(
                    rowsv.at[par * PARTS + j],
                    out_hbm.at[pl.ds((r0 + j) * CHUNK, CHUNK)],
                    ssem[par * PARTS + j]).start()

        pltpu.sync_copy(ids_hbm.at[pl.ds(cbase, PARTS)], idxv[0])
        pltpu.sync_copy(tt_hbm.at[pl.ds(cbase, PARTS)], ttv[0])
        fire_gathers(0, 0)
        pltpu.make_async_copy(
            ids_hbm.at[pl.ds(cbase + PARTS, PARTS)], idxv[1], isem).start()
        pltpu.make_async_copy(
            tt_hbm.at[pl.ds(cbase + PARTS, PARTS)], ttv[1], tsem).start()

        def pair_body(i, c):
            seq_step(i * 2, 0)
            seq_step(i * 2 + 1, 1)
            return c

        lax.fori_loop(0, NSEQ_W // 2, pair_body, 0)

        drain_stores(0)
        drain_stores(1)

    return k


def kernel(input_ids, token_type_ids, word_emb, pos_emb, type_emb, gamma,
           beta):
    ids2d = input_ids.reshape(CHUNKS, CHUNK).astype(jnp.int32)
    tt2d = token_type_ids.reshape(CHUNKS, CHUNK).astype(jnp.int32)
    gb = jnp.stack([gamma, beta]).astype(jnp.float32)
    mesh = plsc.VectorSubcoreMesh(core_axis_name="c", subcore_axis_name="s")
    out = _ln_rows(mesh)(word_emb, ids2d, tt2d, pos_emb, type_emb, gb)
    return out.reshape(B, L, HIDDEN)
